# baseline (speedup 1.0000x reference)
import functools

import jax
import jax.numpy as jnp
from jax import lax
from jax.experimental import pallas as pl
from jax.experimental.pallas import tpu as pltpu
from jax.experimental.pallas import tpu_sc as plsc

_ROW = 128


def _build_gather(B, D, C, NC, NW, per_w):
    n_chunks = per_w // C
    n_pairs = n_chunks // 2
    k_per_c = C // _ROW
    mesh = plsc.VectorSubcoreMesh(core_axis_name="c", subcore_axis_name="s")

    @functools.partial(
        pl.kernel,
        mesh=mesh,
        out_type=jax.ShapeDtypeStruct((B, D), jnp.float32),
        scratch_types=[
            pltpu.VMEM((per_w // _ROW, _ROW), jnp.int32),
            pltpu.VMEM((2, C, D), jnp.float32),
            pltpu.SemaphoreType.DMA,
            pltpu.SemaphoreType.DMA,
        ],
        compiler_params=pltpu.CompilerParams(use_tc_tiling_on_sc=False),
    )
    def gather_kernel(ids_hbm, tab_hbm, out_hbm, idx_v, rows_v, sem0, sem1):
        sems = (sem0, sem1)
        wid = lax.axis_index("s") * NC + lax.axis_index("c")
        base = pl.multiple_of(wid * per_w, 8)

        pltpu.sync_copy(
            ids_hbm.at[pl.ds(base // _ROW, per_w // _ROW), :], idx_v
        )

        def start(i, b):
            for k in range(k_per_c):
                pltpu.async_copy(
                    tab_hbm.at[idx_v.at[i * k_per_c + k]],
                    rows_v.at[b].at[pl.ds(k * _ROW, _ROW)],
                    sems[b],
                )

        def finish(i, b):
            for k in range(k_per_c):
                pltpu.make_async_copy(
                    tab_hbm.at[idx_v.at[0]],
                    rows_v.at[b].at[pl.ds(0, _ROW)],
                    sems[b],
                ).wait()
            off = pl.multiple_of(base + i * C, 8)
            pltpu.sync_copy(rows_v.at[b], out_hbm.at[pl.ds(off, C)])

        start(0, 0)
        start(1, 1)

        def body(j, carry):
            for b in range(2):
                i = j * 2 + b
                finish(i - 2, b)
                start(i, b)
            return carry

        lax.fori_loop(1, n_pairs, body, 0)
        finish(n_chunks - 2, 0)
        finish(n_chunks - 1, 1)

    return gather_kernel


def kernel(token_ids, embedding):
    B0, S = token_ids.shape
    D = embedding.shape[1]
    B = B0 * S
    flat_ids = token_ids.T.reshape(B // _ROW, _ROW).astype(jnp.int32)

    info = plsc.get_sparse_core_info()
    NC, NS = info.num_cores, info.num_subcores
    NW = NC * NS
    per_w = B // NW
    C = 640

    out = _build_gather(B, D, C, NC, NW, per_w)(flat_ids, embedding)
    return out.reshape(S, B0, D).transpose(1, 0, 2)
